# baseline (speedup 1.0000x reference)
import functools

import jax
import jax.numpy as jnp
from jax import lax
from jax.experimental import pallas as pl
from jax.experimental.pallas import tpu as pltpu
from jax.experimental.pallas import tpu_sc as plsc

B = 16384
F = 26
V = 100000
D = 64
H1 = 256
H2 = 128

NC = 2
NS = 16
NW = NC * NS
SPW = B // NW
NBUF = 8

VB = 12544
NJ = 4
VHP = NJ * VB


def _tr_body(xa, xb, out):
    out[...] = jnp.concatenate([xa[0].T, xb[0].T], axis=1)


def _transpose_call(tables_t):
    grid = (F, NJ)
    return pl.pallas_call(
        _tr_body,
        grid=grid,
        in_specs=[
            pl.BlockSpec((1, D, VB), lambda f, j: (f, 0, j)),
            pl.BlockSpec((1, D, VB), lambda f, j: (f, 0, NJ + j)),
        ],
        out_specs=pl.BlockSpec((VB, 128), lambda f, j: (f * NJ + j, 0)),
        out_shape=jax.ShapeDtypeStruct((F * VHP, 128), jnp.float32),
    )(tables_t, tables_t)


def _sc_bi_body(code_hbm, table_hbm, bi_hbm, idx_v, buf_v, out_v,
                sem0, sem1, sem2, sem3, sem4, sem5, sem6, sem7):
    wid = lax.axis_index("s") * NC + lax.axis_index("c")
    base = pl.multiple_of(wid * SPW, SPW)

    sems = (sem0, sem1, sem2, sem3, sem4, sem5, sem6, sem7)

    pltpu.sync_copy(
        code_hbm.at[pl.ds(pl.multiple_of(base * F, 8), SPW * F)],
        idx_v.at[pl.ds(0, SPW * F)],
    )

    def fire(s, b):
        va = idx_v[pl.ds(s * F, 16)]
        vb = idx_v[pl.ds(s * F + 16, 16)]
        for f in range(F):
            lane = va[f] if f < 16 else vb[f - 16]
            tile = lane >> 4
            sub = pl.multiple_of((lane >> 1) & 7, 8)
            pltpu.async_copy(
                table_hbm.at[tile, pl.ds(sub, 1), :],
                buf_v.at[b, f],
                sems[b],
            )

    def drain(b):
        pltpu.make_async_copy(
            table_hbm.at[pl.ds(0, F), pl.ds(0, 1), :],
            buf_v.at[b],
            sems[b],
        ).wait()

    for k in range(NBUF - 1):
        fire(k, k)

    def outer(ss, carry):
        for b in range(NBUF):
            s = ss * NBUF + b
            nxt = s + NBUF - 1

            @pl.when(nxt < SPW)
            def _():
                fire(nxt, (b + NBUF - 1) % NBUF)

            drain(b)
            va = idx_v[pl.ds(s * F, 16)]
            vb2 = idx_v[pl.ds(s * F + 16, 16)]
            pf = [
                ((va[f] if f < 16 else vb2[f - 16]) & 1).astype(jnp.float32)
                for f in range(F)
            ]
            for c in range(D // 16):
                lo = pl.ds(c * 16, 16)
                hi = pl.ds(D + c * 16, 16)
                ve = buf_v[b, 0, 0, lo]
                vo = buf_v[b, 0, 0, hi]
                v = ve + pf[0] * (vo - ve)
                acc = v
                accsq = v * v
                for f in range(1, F):
                    ve = buf_v[b, f, 0, lo]
                    vo = buf_v[b, f, 0, hi]
                    v = ve + pf[f] * (vo - ve)
                    acc = acc + v
                    accsq = accsq + v * v
                out_v[s, pl.ds(c * 16, 16)] = 0.5 * (acc * acc - accsq)

        return carry

    lax.fori_loop(0, SPW // NBUF, outer, 0)

    pltpu.sync_copy(out_v, bi_hbm.at[pl.ds(pl.multiple_of(base, SPW), SPW)])


_sc_bi = functools.partial(
    pl.kernel,
    mesh=plsc.VectorSubcoreMesh(core_axis_name="c", subcore_axis_name="s"),
    compiler_params=pltpu.CompilerParams(use_tc_tiling_on_sc=True),
    out_type=jax.ShapeDtypeStruct((B, D), jnp.float32),
    scratch_types=[
        pltpu.VMEM((SPW * F + 16,), jnp.int32),
        pltpu.VMEM((NBUF, F, 1, 128), jnp.float32),
        pltpu.VMEM((SPW, D), jnp.float32),
        pltpu.SemaphoreType.DMA,
        pltpu.SemaphoreType.DMA,
        pltpu.SemaphoreType.DMA,
        pltpu.SemaphoreType.DMA,
        pltpu.SemaphoreType.DMA,
        pltpu.SemaphoreType.DMA,
        pltpu.SemaphoreType.DMA,
        pltpu.SemaphoreType.DMA,
    ],
)(_sc_bi_body)


BT = 1024


def _mlp_body(bi, xf, w1, b1, w2, b2, w3, wl, c0, out):
    h = jnp.maximum(
        jnp.dot(bi[...], w1[...], preferred_element_type=jnp.float32) + b1[...], 0.0
    )
    h = jnp.maximum(
        jnp.dot(h, w2[...], preferred_element_type=jnp.float32) + b2[...], 0.0
    )
    deep = jnp.dot(h, w3[...], preferred_element_type=jnp.float32)
    lin = jnp.dot(xf[...], wl[...], preferred_element_type=jnp.float32)
    out[...] = jax.nn.sigmoid(deep + lin + c0[...])


def _mlp_call(bi, xf, w1, b1, w2, b2, w3, wl, c0):
    grid = (B // BT,)
    return pl.pallas_call(
        _mlp_body,
        grid=grid,
        in_specs=[
            pl.BlockSpec((BT, D), lambda i: (i, 0)),
            pl.BlockSpec((BT, F), lambda i: (i, 0)),
            pl.BlockSpec((D, H1), lambda i: (0, 0)),
            pl.BlockSpec((1, H1), lambda i: (0, 0)),
            pl.BlockSpec((H1, H2), lambda i: (0, 0)),
            pl.BlockSpec((1, H2), lambda i: (0, 0)),
            pl.BlockSpec((H2, 1), lambda i: (0, 0)),
            pl.BlockSpec((F, 1), lambda i: (0, 0)),
            pl.BlockSpec((1, 1), lambda i: (0, 0)),
        ],
        out_specs=pl.BlockSpec((BT, 1), lambda i: (i, 0)),
        out_shape=jax.ShapeDtypeStruct((B, 1), jnp.float32),
    )(bi, xf, w1, b1, w2, b2, w3, wl, c0)


def kernel(x, tables, Wl, bl, W1, b1, W2, b2, W3, b3):
    x = x.astype(jnp.int32)
    table2 = _transpose_call(jnp.transpose(tables, (0, 2, 1)))
    code = (x % VHP + (jnp.arange(F, dtype=jnp.int32) * VHP)[None, :]) * 2 + (
        x // VHP
    )
    bi = _sc_bi(code.reshape(B * F), table2.reshape(F * VHP // 8, 8, 128))
    xf = x.astype(jnp.float32)
    c0 = (bl + b3).reshape(1, 1)
    return _mlp_call(
        bi,
        xf,
        W1,
        b1.reshape(1, H1),
        W2,
        b2.reshape(1, H2),
        W3,
        Wl,
        c0,
    )


# speedup vs baseline: 1.2169x; 1.2169x over previous
import functools

import jax
import jax.numpy as jnp
from jax import lax
from jax.experimental import pallas as pl
from jax.experimental.pallas import tpu as pltpu
from jax.experimental.pallas import tpu_sc as plsc

B = 16384
F = 26
V = 100000
D = 64
H1 = 256
H2 = 128

NC = 2
NS = 16
NW = NC * NS
SPW = B // NW
NBUF = 4

VB = 12544
NJ = 4
VHP = NJ * VB


def _tr_body(xa, xb, out):
    out[...] = jnp.concatenate([xa[0].T, xb[0].T], axis=1)


def _transpose_call(tables_t):
    grid = (F, NJ)
    return pl.pallas_call(
        _tr_body,
        grid=grid,
        in_specs=[
            pl.BlockSpec((1, D, VB), lambda f, j: (f, 0, j)),
            pl.BlockSpec((1, D, VB), lambda f, j: (f, 0, NJ + j)),
        ],
        out_specs=pl.BlockSpec((VB, 128), lambda f, j: (f * NJ + j, 0)),
        out_shape=jax.ShapeDtypeStruct((F * VHP, 128), jnp.float32),
    )(tables_t, tables_t)


def _sc_bi_body(code_hbm, table_hbm, bi_hbm, idx_v, buf_v, out_v,
                sem0, sem1, sem2, sem3):
    wid = lax.axis_index("s") * NC + lax.axis_index("c")
    base = pl.multiple_of(wid * SPW, SPW)

    sems = (sem0, sem1, sem2, sem3)

    pltpu.sync_copy(
        code_hbm.at[pl.ds(pl.multiple_of(base * F, 8), SPW * F)],
        idx_v.at[pl.ds(0, SPW * F)],
    )

    def fire(s, b):
        va = idx_v[pl.ds(s * F, 16)]
        vb = idx_v[pl.ds(s * F + 16, 16)]
        for f in range(F):
            lane = va[f] if f < 16 else vb[f - 16]
            tile = lane >> 4
            sub = pl.multiple_of((lane >> 1) & 7, 8)
            pltpu.async_copy(
                table_hbm.at[tile, pl.ds(sub, 1), :],
                buf_v.at[b, f],
                sems[b],
            )

    def drain(b):
        pltpu.make_async_copy(
            table_hbm.at[pl.ds(0, F), pl.ds(0, 1), :],
            buf_v.at[b],
            sems[b],
        ).wait()

    for k in range(NBUF - 1):
        fire(k, k)

    def outer(ss, carry):
        for b in range(NBUF):
            s = ss * NBUF + b
            nxt = s + NBUF - 1

            @pl.when(nxt < SPW)
            def _():
                fire(nxt, (b + NBUF - 1) % NBUF)

            drain(b)
            va = idx_v[pl.ds(s * F, 16)]
            vb2 = idx_v[pl.ds(s * F + 16, 16)]
            pf = [
                ((va[f] if f < 16 else vb2[f - 16]) & 1).astype(jnp.float32)
                for f in range(F)
            ]
            for c in range(D // 16):
                lo = pl.ds(c * 16, 16)
                hi = pl.ds(D + c * 16, 16)
                ve = buf_v[b, 0, 0, lo]
                vo = buf_v[b, 0, 0, hi]
                v = ve + pf[0] * (vo - ve)
                acc = v
                accsq = v * v
                for f in range(1, F):
                    ve = buf_v[b, f, 0, lo]
                    vo = buf_v[b, f, 0, hi]
                    v = ve + pf[f] * (vo - ve)
                    acc = acc + v
                    accsq = accsq + v * v
                out_v[s, pl.ds(c * 16, 16)] = 0.5 * (acc * acc - accsq)

        return carry

    lax.fori_loop(0, SPW // NBUF, outer, 0)

    pltpu.sync_copy(out_v, bi_hbm.at[pl.ds(pl.multiple_of(base, SPW), SPW)])


_sc_bi = functools.partial(
    pl.kernel,
    mesh=plsc.VectorSubcoreMesh(core_axis_name="c", subcore_axis_name="s"),
    compiler_params=pltpu.CompilerParams(use_tc_tiling_on_sc=True),
    out_type=jax.ShapeDtypeStruct((B, D), jnp.float32),
    scratch_types=[
        pltpu.VMEM((SPW * F + 16,), jnp.int32),
        pltpu.VMEM((NBUF, F, 1, 128), jnp.float32),
        pltpu.VMEM((SPW, D), jnp.float32),
        pltpu.SemaphoreType.DMA,
        pltpu.SemaphoreType.DMA,
        pltpu.SemaphoreType.DMA,
        pltpu.SemaphoreType.DMA,
    ],
)(_sc_bi_body)


BT = 1024


def _mlp_body(bi, xf, w1, b1, w2, b2, w3, wl, c0, out):
    h = jnp.maximum(
        jnp.dot(bi[...], w1[...], preferred_element_type=jnp.float32) + b1[...], 0.0
    )
    h = jnp.maximum(
        jnp.dot(h, w2[...], preferred_element_type=jnp.float32) + b2[...], 0.0
    )
    deep = jnp.dot(h, w3[...], preferred_element_type=jnp.float32)
    lin = jnp.dot(xf[...], wl[...], preferred_element_type=jnp.float32)
    out[...] = jax.nn.sigmoid(deep + lin + c0[...])


def _mlp_call(bi, xf, w1, b1, w2, b2, w3, wl, c0):
    grid = (B // BT,)
    return pl.pallas_call(
        _mlp_body,
        grid=grid,
        in_specs=[
            pl.BlockSpec((BT, D), lambda i: (i, 0)),
            pl.BlockSpec((BT, F), lambda i: (i, 0)),
            pl.BlockSpec((D, H1), lambda i: (0, 0)),
            pl.BlockSpec((1, H1), lambda i: (0, 0)),
            pl.BlockSpec((H1, H2), lambda i: (0, 0)),
            pl.BlockSpec((1, H2), lambda i: (0, 0)),
            pl.BlockSpec((H2, 1), lambda i: (0, 0)),
            pl.BlockSpec((F, 1), lambda i: (0, 0)),
            pl.BlockSpec((1, 1), lambda i: (0, 0)),
        ],
        out_specs=pl.BlockSpec((BT, 1), lambda i: (i, 0)),
        out_shape=jax.ShapeDtypeStruct((B, 1), jnp.float32),
    )(bi, xf, w1, b1, w2, b2, w3, wl, c0)


def kernel(x, tables, Wl, bl, W1, b1, W2, b2, W3, b3):
    x = x.astype(jnp.int32)
    table2 = _transpose_call(jnp.transpose(tables, (0, 2, 1)))
    code = (x % VHP + (jnp.arange(F, dtype=jnp.int32) * VHP)[None, :]) * 2 + (
        x // VHP
    )
    bi = _sc_bi(code.reshape(B * F), table2.reshape(F * VHP // 8, 8, 128))
    xf = x.astype(jnp.float32)
    c0 = (bl + b3).reshape(1, 1)
    return _mlp_call(
        bi,
        xf,
        W1,
        b1.reshape(1, H1),
        W2,
        b2.reshape(1, H2),
        W3,
        Wl,
        c0,
    )
